# baseline (speedup 1.0000x reference)
import functools
import jax
import jax.numpy as jnp
from jax import lax
from jax.experimental import pallas as pl
from jax.experimental.pallas import tpu as pltpu
from jax.experimental.pallas import tpu_sc as plsc

H = 8
D = 16
OUT = 128
IN = 128
N = 10000
E = 320000

NC = 2
NS = 16
LANES = 16

ACC = OUT + LANES
C = 32
EPW = E // NS
CHUNKS = EPW // C
NPAD = 10240
ROWS_PER_TILE = NPAD // NS


def _pre_body(x_ref, w_ref, b_ref, q_ref, kv_ref):
    y = jnp.dot(x_ref[...], w_ref[...], preferred_element_type=jnp.float32)
    y = y + b_ref[...]
    q_ref[...] = y[:, 0:OUT]
    kv_ref[...] = y[:, OUT:3 * OUT]


def _tc_pre(x, w, b):
    BN = 1000
    grid = (N // BN,)
    return pl.pallas_call(
        _pre_body,
        grid=grid,
        in_specs=[
            pl.BlockSpec((BN, IN), lambda i: (i, 0)),
            pl.BlockSpec((IN, 3 * OUT), lambda i: (0, 0)),
            pl.BlockSpec((1, 3 * OUT), lambda i: (0, 0)),
        ],
        out_specs=[
            pl.BlockSpec((BN, OUT), lambda i: (i, 0)),
            pl.BlockSpec((BN, 2 * OUT), lambda i: (i, 0)),
        ],
        out_shape=[
            jax.ShapeDtypeStruct((N, OUT), jnp.float32),
            jax.ShapeDtypeStruct((N, 2 * OUT), jnp.float32),
        ],
    )(x, w, b)


def _post_body(acc_ref, x_ref, w_ref, b_ref, al_ref, g_ref, be_ref, o_ref):
    acc = acc_ref[...]
    agg = acc[:, 0:OUT]
    den = acc[:, OUT:OUT + H] + 1e-16
    bn = agg.shape[0]
    aggn = (agg.reshape(bn, H, D) / den[:, :, None]).reshape(bn, OUT)
    o = jnp.dot(jax.nn.gelu(aggn), w_ref[...],
                preferred_element_type=jnp.float32) + b_ref[...]
    al = al_ref[...]
    r = al * o + (1.0 - al) * x_ref[...]
    r = jnp.maximum(r, 0.0)
    mu = jnp.mean(r, axis=-1, keepdims=True)
    var = jnp.mean((r - mu) ** 2, axis=-1, keepdims=True)
    rn = (r - mu) * lax.rsqrt(var + 1e-5)
    o_ref[...] = g_ref[...] * rn + be_ref[...]


def _tc_post(acc, x, w, b, al, gamma, beta):
    BN = 1000
    grid = (N // BN,)
    return pl.pallas_call(
        _post_body,
        grid=grid,
        in_specs=[
            pl.BlockSpec((BN, ACC), lambda i: (i, 0)),
            pl.BlockSpec((BN, IN), lambda i: (i, 0)),
            pl.BlockSpec((OUT, OUT), lambda i: (0, 0)),
            pl.BlockSpec((1, OUT), lambda i: (0, 0)),
            pl.BlockSpec((1, 1), lambda i: (0, 0)),
            pl.BlockSpec((1, OUT), lambda i: (0, 0)),
            pl.BlockSpec((1, OUT), lambda i: (0, 0)),
        ],
        out_specs=pl.BlockSpec((BN, OUT), lambda i: (i, 0)),
        out_shape=jax.ShapeDtypeStruct((N, OUT), jnp.float32),
    )(acc, x, w, b, al, gamma, beta)


def _edge_chunks(tid, q_hbm, kv_hbm, src_hbm, dst_hbm,
                 idxs0, idxd0, idxs1, idxd1, qb0, kvb0, qb1, kvb1,
                 mwb, acc_sh, sem_a, sem_b):
    base0 = tid * EPW

    lanes = lax.iota(jnp.int32, LANES)

    def compute_scatter(qb, kvb, idxd):
        def edge(e, _):
            wvec = jnp.zeros((LANES,), jnp.float32)
            for h in range(H):
                prod = qb[e, pl.ds(h * D, D)] * kvb[e, pl.ds(h * D, D)]
                sh = jnp.sum(prod)
                wsp = jnp.exp(jnp.full((LANES,), sh, jnp.float32))
                mwb[e, pl.ds(h * D, D)] = kvb[e, pl.ds(OUT + h * D, D)] * wsp
                wvec = jnp.where(lanes == h, wsp, wvec)
            mwb[e, pl.ds(OUT, LANES)] = wvec
            return 0

        lax.fori_loop(0, C, edge, 0)
        pltpu.sync_copy(mwb, acc_sh.at[idxd], add=True)

    pltpu.sync_copy(src_hbm.at[pl.ds(base0, C)], idxs0)
    pltpu.sync_copy(dst_hbm.at[pl.ds(base0, C)], idxd0)
    pltpu.async_copy(q_hbm.at[idxd0], qb0, sem_a)
    pltpu.async_copy(kv_hbm.at[idxs0], kvb0, sem_a)

    def pair(i, _):
        b1 = base0 + (2 * i + 1) * C
        pltpu.sync_copy(src_hbm.at[pl.ds(b1, C)], idxs1)
        pltpu.sync_copy(dst_hbm.at[pl.ds(b1, C)], idxd1)
        pltpu.async_copy(q_hbm.at[idxd1], qb1, sem_b)
        pltpu.async_copy(kv_hbm.at[idxs1], kvb1, sem_b)
        pltpu.make_async_copy(q_hbm.at[idxd0], qb0, sem_a).wait()
        pltpu.make_async_copy(kv_hbm.at[idxs0], kvb0, sem_a).wait()
        compute_scatter(qb0, kvb0, idxd0)
        b2 = base0 + (2 * i + 2) * C
        pltpu.sync_copy(src_hbm.at[pl.ds(b2, C)], idxs0)
        pltpu.sync_copy(dst_hbm.at[pl.ds(b2, C)], idxd0)
        pltpu.async_copy(q_hbm.at[idxd0], qb0, sem_a)
        pltpu.async_copy(kv_hbm.at[idxs0], kvb0, sem_a)
        pltpu.make_async_copy(q_hbm.at[idxd1], qb1, sem_b).wait()
        pltpu.make_async_copy(kv_hbm.at[idxs1], kvb1, sem_b).wait()
        compute_scatter(qb1, kvb1, idxd1)
        return 0

    lax.fori_loop(0, (CHUNKS - 1) // 2, pair, 0)

    pltpu.make_async_copy(q_hbm.at[idxd0], qb0, sem_a).wait()
    pltpu.make_async_copy(kv_hbm.at[idxs0], kvb0, sem_a).wait()
    compute_scatter(qb0, kvb0, idxd0)


def _sc_body(q0, kv0, q1, kv1, s0, d0, s1, d1,
             acc0, acc1,
             idxs0, idxd0, idxs1, idxd1, qb0, kvb0, qb1, kvb1,
             mwb, acc_sh, sem_a, sem_b):
    cid = lax.axis_index("c")
    tid = lax.axis_index("s")

    zv = jnp.zeros((LANES,), jnp.float32)

    def zero_mwb(i, _):
        r = i // (ACC // LANES)
        cc = i % (ACC // LANES)
        mwb[r, pl.ds(cc * LANES, LANES)] = zv
        return 0

    lax.fori_loop(0, C * (ACC // LANES), zero_mwb, 0)

    def zero_stripe(j, _):
        r0z = tid * ROWS_PER_TILE + j * C
        pltpu.sync_copy(mwb, acc_sh.at[pl.ds(r0z, C)])
        return 0

    lax.fori_loop(0, ROWS_PER_TILE // C, zero_stripe, 0)

    plsc.subcore_barrier()

    @pl.when(cid == 0)
    def _():
        _edge_chunks(tid, q0, kv0, s0, d0,
                     idxs0, idxd0, idxs1, idxd1, qb0, kvb0, qb1, kvb1,
                     mwb, acc_sh, sem_a, sem_b)

    @pl.when(cid == 1)
    def _():
        _edge_chunks(tid, q1, kv1, s1, d1,
                     idxs0, idxd0, idxs1, idxd1, qb0, kvb0, qb1, kvb1,
                     mwb, acc_sh, sem_a, sem_b)

    plsc.subcore_barrier()

    r0 = tid * ROWS_PER_TILE

    @pl.when(cid == 0)
    def _():
        pltpu.sync_copy(acc_sh.at[pl.ds(r0, ROWS_PER_TILE)],
                        acc0.at[pl.ds(r0, ROWS_PER_TILE)])

    @pl.when(cid == 1)
    def _():
        pltpu.sync_copy(acc_sh.at[pl.ds(r0, ROWS_PER_TILE)],
                        acc1.at[pl.ds(r0, ROWS_PER_TILE)])


_sc_edge = functools.partial(
    pl.kernel,
    out_type=[
        jax.ShapeDtypeStruct((NPAD, ACC), jnp.float32),
        jax.ShapeDtypeStruct((NPAD, ACC), jnp.float32),
    ],
    mesh=plsc.VectorSubcoreMesh(core_axis_name="c", subcore_axis_name="s",
                                num_cores=NC, num_subcores=NS),
    compiler_params=pltpu.CompilerParams(needs_layout_passes=False,
                                         use_tc_tiling_on_sc=False),
    scratch_types=[
        pltpu.VMEM((C,), jnp.int32),
        pltpu.VMEM((C,), jnp.int32),
        pltpu.VMEM((C,), jnp.int32),
        pltpu.VMEM((C,), jnp.int32),
        pltpu.VMEM((C, OUT), jnp.float32),
        pltpu.VMEM((C, 2 * OUT), jnp.float32),
        pltpu.VMEM((C, OUT), jnp.float32),
        pltpu.VMEM((C, 2 * OUT), jnp.float32),
        pltpu.VMEM((C, ACC), jnp.float32),
        pltpu.VMEM_SHARED((NPAD, ACC), jnp.float32),
        pltpu.SemaphoreType.DMA,
        pltpu.SemaphoreType.DMA,
    ],
)(_sc_body)


def _fold_weights(W_kqv, b_kqv, p_q, a_k, m_v):
    Wk, Wq, Wv = jnp.split(W_kqv, 3, axis=1)
    bk, bq, bv = jnp.split(b_kqv, 3)
    sc = (p_q / jnp.sqrt(jnp.float32(D)))
    Wq2 = (Wq.reshape(IN, H, D) * sc[None, :, None]).reshape(IN, OUT)
    bq2 = (bq.reshape(H, D) * sc[:, None]).reshape(OUT)
    Wk2 = jnp.einsum('ihd,hde->ihe', Wk.reshape(IN, H, D), a_k).reshape(IN, OUT)
    bk2 = jnp.einsum('hd,hde->he', bk.reshape(H, D), a_k).reshape(OUT)
    Wv2 = jnp.einsum('ihd,hde->ihe', Wv.reshape(IN, H, D), m_v).reshape(IN, OUT)
    bv2 = jnp.einsum('hd,hde->he', bv.reshape(H, D), m_v).reshape(OUT)
    W = jnp.concatenate([Wq2, Wk2, Wv2], axis=1)
    b = jnp.concatenate([bq2, bk2, bv2])[None, :]
    return W, b


def kernel(x_paper, x_author, edge_index_writes, edge_index_rev,
           W_kqv_paper, b_kqv_paper, W_kqv_author, b_kqv_author,
           a_writes, m_writes, p_writes, a_rev, m_rev, p_rev,
           W_out_paper, b_out_paper, W_out_author, b_out_author,
           skip_paper, skip_author, ln_gamma, ln_beta):
    W_p, b_p = _fold_weights(W_kqv_paper, b_kqv_paper, p_writes, a_rev, m_rev)
    W_a, b_a = _fold_weights(W_kqv_author, b_kqv_author, p_rev,
                             a_writes, m_writes)

    q_p, kv_p = _tc_pre(x_paper, W_p, b_p)
    q_a, kv_a = _tc_pre(x_author, W_a, b_a)

    acc_p, acc_a = _sc_edge(
        q_p, kv_a, q_a, kv_p,
        edge_index_writes[0], edge_index_writes[1],
        edge_index_rev[0], edge_index_rev[1])

    al_p = jax.nn.sigmoid(skip_paper).reshape(1, 1)
    al_a = jax.nn.sigmoid(skip_author).reshape(1, 1)
    gamma = ln_gamma[None, :]
    beta = ln_beta[None, :]
    b_out_p = b_out_paper[None, :]
    b_out_a = b_out_author[None, :]

    out_p = _tc_post(acc_p, x_paper, W_out_paper, b_out_p,
                     al_p, gamma, beta)
    out_a = _tc_post(acc_a, x_author, W_out_author,
                     b_out_a, al_a, gamma, beta)
    return out_p, out_a
